# baseline (device time: 52746 ns/iter reference)
import jax
import jax.numpy as jnp
from jax import lax
from jax.experimental import pallas as pl
from jax.experimental.pallas import tpu as pltpu

N_DEV = 8


def kernel(x, w_mat, scale_x, scale_w):
    m_per, k = x.shape
    n = w_mat.shape[1]
    n_per = n // N_DEV
    m_tot = m_per * N_DEV

    def body(x_ref, w_ref, sx_ref, sw_ref, out_ref,
             xb_ref, send_ref, comm_ref, send_sems, recv_sems):
        my = lax.axis_index("i")

        barrier = pltpu.get_barrier_semaphore()
        for p in range(N_DEV):
            @pl.when(my != p)
            def _():
                pl.semaphore_signal(
                    barrier, inc=1, device_id=(p,),
                    device_id_type=pl.DeviceIdType.MESH,
                )
        pl.semaphore_wait(barrier, N_DEV - 1)

        s = sx_ref[0] * sw_ref[0]
        xb_ref[...] = x_ref[...].astype(jnp.bfloat16)

        def send_to(j):
            return pltpu.make_async_remote_copy(
                src_ref=send_ref.at[j],
                dst_ref=comm_ref.at[my],
                send_sem=send_sems.at[j],
                recv_sem=recv_sems.at[my],
                device_id=(j,),
                device_id_type=pl.DeviceIdType.MESH,
            )

        def recv_from(q):
            return pltpu.make_async_remote_copy(
                src_ref=send_ref.at[q],
                dst_ref=comm_ref.at[q],
                send_sem=send_sems.at[q],
                recv_sem=recv_sems.at[q],
                device_id=(q,),
                device_id_type=pl.DeviceIdType.MESH,
            )

        for j in range(N_DEV):
            wb = w_ref[:, j * n_per:(j + 1) * n_per].astype(jnp.bfloat16)
            acc = jnp.dot(xb_ref[...], wb, preferred_element_type=jnp.float32)
            y = acc * s
            y = y / (1.0 + jnp.exp(-jnp.clip(y, -60.0, 60.0)))

            @pl.when(my == j)
            def _():
                out_ref[j * m_per:(j + 1) * m_per, :] = y

            @pl.when(my != j)
            def _():
                send_ref[j] = y.astype(jnp.bfloat16)
                send_to(j).start()

        for q in range(N_DEV):
            @pl.when(my != q)
            def _():
                recv_from(q).wait_recv()
                out_ref[q * m_per:(q + 1) * m_per, :] = (
                    comm_ref[q].astype(jnp.float32))

        for j in range(N_DEV):
            @pl.when(my != j)
            def _():
                send_to(j).wait_send()

    return pl.pallas_call(
        body,
        out_shape=jax.ShapeDtypeStruct((m_tot, n_per), jnp.float32),
        in_specs=[
            pl.BlockSpec(memory_space=pltpu.VMEM),
            pl.BlockSpec(memory_space=pltpu.VMEM),
            pl.BlockSpec(memory_space=pltpu.SMEM),
            pl.BlockSpec(memory_space=pltpu.SMEM),
        ],
        out_specs=pl.BlockSpec(memory_space=pltpu.VMEM),
        scratch_shapes=[
            pltpu.VMEM((m_per, k), jnp.bfloat16),
            pltpu.VMEM((N_DEV, m_per, n_per), jnp.bfloat16),
            pltpu.VMEM((N_DEV, m_per, n_per), jnp.bfloat16),
            pltpu.SemaphoreType.DMA((N_DEV,)),
            pltpu.SemaphoreType.DMA((N_DEV,)),
        ],
        compiler_params=pltpu.CompilerParams(
            collective_id=0,
            vmem_limit_bytes=100 * 1024 * 1024,
        ),
    )(x, w_mat, scale_x, scale_w)


# device time: 45798 ns/iter; 1.1517x vs baseline; 1.1517x over previous
import jax
import jax.numpy as jnp
from jax import lax
from jax.experimental import pallas as pl
from jax.experimental.pallas import tpu as pltpu

N_DEV = 8


def kernel(x, w_mat, scale_x, scale_w):
    m_per, k = x.shape
    n = w_mat.shape[1]
    n_per = n // N_DEV
    m_tot = m_per * N_DEV

    def body(x_ref, w_ref, sx_ref, sw_ref, out_ref,
             xb_ref, send_ref, comm_ref, send_sems, recv_sems):
        my = lax.axis_index("i")

        barrier = pltpu.get_barrier_semaphore()
        for p in range(N_DEV):
            @pl.when(my != p)
            def _():
                pl.semaphore_signal(
                    barrier, inc=1, device_id=(p,),
                    device_id_type=pl.DeviceIdType.MESH,
                )
        pl.semaphore_wait(barrier, N_DEV - 1)

        s = sx_ref[0] * sw_ref[0]
        xb_ref[...] = x_ref[...].astype(jnp.bfloat16)

        def chunk(col_off):
            wb = w_ref[:, pl.ds(col_off, n_per)].astype(jnp.bfloat16)
            acc = jnp.dot(xb_ref[...], wb, preferred_element_type=jnp.float32)
            y = acc * s
            return y / (1.0 + jnp.exp(-jnp.clip(y, -60.0, 60.0)))

        rdmas = []
        for st in range(N_DEV - 1):
            j = lax.rem(my + 1 + st, N_DEV)
            send_ref[st] = chunk(j * n_per).astype(jnp.bfloat16)
            rdma = pltpu.make_async_remote_copy(
                src_ref=send_ref.at[st],
                dst_ref=comm_ref.at[st],
                send_sem=send_sems.at[st],
                recv_sem=recv_sems.at[st],
                device_id=(j,),
                device_id_type=pl.DeviceIdType.MESH,
            )
            rdma.start()
            rdmas.append(rdma)

        out_ref[pl.ds(my * m_per, m_per), :] = chunk(my * n_per)

        for st in range(N_DEV - 1):
            rdmas[st].wait_recv()
            src = lax.rem(my - 1 - st + N_DEV, N_DEV)
            out_ref[pl.ds(src * m_per, m_per), :] = (
                comm_ref[st].astype(jnp.float32))

        for st in range(N_DEV - 1):
            rdmas[st].wait_send()

    return pl.pallas_call(
        body,
        out_shape=jax.ShapeDtypeStruct((m_tot, n_per), jnp.float32),
        in_specs=[
            pl.BlockSpec(memory_space=pltpu.VMEM),
            pl.BlockSpec(memory_space=pltpu.VMEM),
            pl.BlockSpec(memory_space=pltpu.SMEM),
            pl.BlockSpec(memory_space=pltpu.SMEM),
        ],
        out_specs=pl.BlockSpec(memory_space=pltpu.VMEM),
        scratch_shapes=[
            pltpu.VMEM((m_per, k), jnp.bfloat16),
            pltpu.VMEM((N_DEV - 1, m_per, n_per), jnp.bfloat16),
            pltpu.VMEM((N_DEV - 1, m_per, n_per), jnp.bfloat16),
            pltpu.SemaphoreType.DMA((N_DEV - 1,)),
            pltpu.SemaphoreType.DMA((N_DEV - 1,)),
        ],
        compiler_params=pltpu.CompilerParams(
            collective_id=0,
            vmem_limit_bytes=100 * 1024 * 1024,
        ),
    )(x, w_mat, scale_x, scale_w)


# device time: 44986 ns/iter; 1.1725x vs baseline; 1.0181x over previous
import jax
import jax.numpy as jnp
from jax import lax
from jax.experimental import pallas as pl
from jax.experimental.pallas import tpu as pltpu

N_DEV = 8


def kernel(x, w_mat, scale_x, scale_w):
    m_per, k = x.shape
    n = w_mat.shape[1]
    n_per = n // N_DEV
    m_tot = m_per * N_DEV

    def body(x_ref, w_ref, sx_ref, sw_ref, out_ref,
             xb_ref, send_ref, comm_ref, ssc_ref, rsc_ref,
             send_sems, recv_sems, ssc_sems, rsc_sems):
        my = lax.axis_index("i")

        barrier = pltpu.get_barrier_semaphore()
        for p in range(N_DEV):
            @pl.when(my != p)
            def _():
                pl.semaphore_signal(
                    barrier, inc=1, device_id=(p,),
                    device_id_type=pl.DeviceIdType.MESH,
                )
        pl.semaphore_wait(barrier, N_DEV - 1)

        s = sx_ref[0] * sw_ref[0]
        xb_ref[...] = x_ref[...].astype(jnp.bfloat16)

        def chunk(col_off):
            wb = w_ref[:, pl.ds(col_off, n_per)].astype(jnp.bfloat16)
            acc = jnp.dot(xb_ref[...], wb, preferred_element_type=jnp.float32)
            y = acc * s
            return y / (1.0 + jnp.exp(-jnp.clip(y, -60.0, 60.0)))

        rdmas = []
        for st in range(N_DEV - 1):
            j = lax.rem(my + 1 + st, N_DEV)
            y = chunk(j * n_per)
            amax = jnp.max(jnp.abs(y), axis=0, keepdims=True)
            qscale = jnp.maximum(amax, 1e-30) / 127.0
            send_ref[st] = jnp.round(y / qscale).astype(jnp.int8)
            ssc_ref[st] = qscale
            data = pltpu.make_async_remote_copy(
                src_ref=send_ref.at[st],
                dst_ref=comm_ref.at[st],
                send_sem=send_sems.at[st],
                recv_sem=recv_sems.at[st],
                device_id=(j,),
                device_id_type=pl.DeviceIdType.MESH,
            )
            scales = pltpu.make_async_remote_copy(
                src_ref=ssc_ref.at[st],
                dst_ref=rsc_ref.at[st],
                send_sem=ssc_sems.at[st],
                recv_sem=rsc_sems.at[st],
                device_id=(j,),
                device_id_type=pl.DeviceIdType.MESH,
            )
            data.start()
            scales.start()
            rdmas.append((data, scales))

        out_ref[pl.ds(my * m_per, m_per), :] = chunk(my * n_per)

        for st in range(N_DEV - 1):
            data, scales = rdmas[st]
            data.wait_recv()
            scales.wait_recv()
            src = lax.rem(my - 1 - st + N_DEV, N_DEV)
            deq = comm_ref[st].astype(jnp.float32) * rsc_ref[st]
            out_ref[pl.ds(src * m_per, m_per), :] = deq

        for st in range(N_DEV - 1):
            data, scales = rdmas[st]
            data.wait_send()
            scales.wait_send()

    return pl.pallas_call(
        body,
        out_shape=jax.ShapeDtypeStruct((m_tot, n_per), jnp.float32),
        in_specs=[
            pl.BlockSpec(memory_space=pltpu.VMEM),
            pl.BlockSpec(memory_space=pltpu.VMEM),
            pl.BlockSpec(memory_space=pltpu.SMEM),
            pl.BlockSpec(memory_space=pltpu.SMEM),
        ],
        out_specs=pl.BlockSpec(memory_space=pltpu.VMEM),
        scratch_shapes=[
            pltpu.VMEM((m_per, k), jnp.bfloat16),
            pltpu.VMEM((N_DEV - 1, m_per, n_per), jnp.int8),
            pltpu.VMEM((N_DEV - 1, m_per, n_per), jnp.int8),
            pltpu.VMEM((N_DEV - 1, 1, n_per), jnp.float32),
            pltpu.VMEM((N_DEV - 1, 1, n_per), jnp.float32),
            pltpu.SemaphoreType.DMA((N_DEV - 1,)),
            pltpu.SemaphoreType.DMA((N_DEV - 1,)),
            pltpu.SemaphoreType.DMA((N_DEV - 1,)),
            pltpu.SemaphoreType.DMA((N_DEV - 1,)),
        ],
        compiler_params=pltpu.CompilerParams(
            collective_id=0,
            vmem_limit_bytes=100 * 1024 * 1024,
        ),
    )(x, w_mat, scale_x, scale_w)


# device time: 37642 ns/iter; 1.4013x vs baseline; 1.1951x over previous
import jax
import jax.numpy as jnp
from jax import lax
from jax.experimental import pallas as pl
from jax.experimental.pallas import tpu as pltpu

N_DEV = 8


def kernel(x, w_mat, scale_x, scale_w):
    m_per, k = x.shape
    n = w_mat.shape[1]
    n_per = n // N_DEV
    m_tot = m_per * N_DEV

    def body(x_ref, w_ref, sx_ref, sw_ref, out_ref,
             xb_ref, wv_ref, send_ref, comm_ref, ssc_ref, rsc_ref,
             wdma_sems, send_sems, recv_sems, ssc_sems, rsc_sems):
        my = lax.axis_index("i")

        barrier = pltpu.get_barrier_semaphore()
        for p in range(N_DEV):
            @pl.when(my != p)
            def _():
                pl.semaphore_signal(
                    barrier, inc=1, device_id=(p,),
                    device_id_type=pl.DeviceIdType.MESH,
                )
        pl.semaphore_wait(barrier, N_DEV - 1)

        s = sx_ref[0] * sw_ref[0]

        col = [lax.rem(my + 1 + st, N_DEV) * n_per for st in range(N_DEV)]

        def wcopy(st, buf):
            return pltpu.make_async_copy(
                w_ref.at[:, pl.ds(col[st], n_per)],
                wv_ref.at[buf],
                wdma_sems.at[buf],
            )

        wcopy(0, 0).start()
        xb_ref[...] = x_ref[...].astype(jnp.bfloat16)

        rdmas = []
        for st in range(N_DEV):
            buf = st % 2
            if st + 1 < N_DEV:
                wcopy(st + 1, 1 - buf).start()
            wcopy(st, buf).wait()
            wb = wv_ref[buf].astype(jnp.bfloat16)
            acc = jnp.dot(xb_ref[...], wb, preferred_element_type=jnp.float32)
            y = acc * s
            y = y / (1.0 + jnp.exp(-jnp.clip(y, -60.0, 60.0)))

            if st == N_DEV - 1:
                out_ref[pl.ds(my * m_per, m_per), :] = y
                break

            j = lax.rem(my + 1 + st, N_DEV)
            amax = jnp.max(jnp.abs(y), axis=0, keepdims=True)
            qscale = jnp.maximum(amax, 1e-30) / 127.0
            send_ref[st] = jnp.round(y / qscale).astype(jnp.int8)
            ssc_ref[st] = qscale
            data = pltpu.make_async_remote_copy(
                src_ref=send_ref.at[st],
                dst_ref=comm_ref.at[st],
                send_sem=send_sems.at[st],
                recv_sem=recv_sems.at[st],
                device_id=(j,),
                device_id_type=pl.DeviceIdType.MESH,
            )
            scales = pltpu.make_async_remote_copy(
                src_ref=ssc_ref.at[st],
                dst_ref=rsc_ref.at[st],
                send_sem=ssc_sems.at[st],
                recv_sem=rsc_sems.at[st],
                device_id=(j,),
                device_id_type=pl.DeviceIdType.MESH,
            )
            data.start()
            scales.start()
            rdmas.append((data, scales))

        for st in range(N_DEV - 1):
            data, scales = rdmas[st]
            data.wait_recv()
            scales.wait_recv()
            src = lax.rem(my - 1 - st + N_DEV, N_DEV)
            deq = comm_ref[st].astype(jnp.float32) * rsc_ref[st]
            out_ref[pl.ds(src * m_per, m_per), :] = deq

        for st in range(N_DEV - 1):
            data, scales = rdmas[st]
            data.wait_send()
            scales.wait_send()

    return pl.pallas_call(
        body,
        out_shape=jax.ShapeDtypeStruct((m_tot, n_per), jnp.float32),
        in_specs=[
            pl.BlockSpec(memory_space=pltpu.VMEM),
            pl.BlockSpec(memory_space=pltpu.MemorySpace.HBM),
            pl.BlockSpec(memory_space=pltpu.SMEM),
            pl.BlockSpec(memory_space=pltpu.SMEM),
        ],
        out_specs=pl.BlockSpec(memory_space=pltpu.VMEM),
        scratch_shapes=[
            pltpu.VMEM((m_per, k), jnp.bfloat16),
            pltpu.VMEM((2, k, n_per), jnp.float32),
            pltpu.VMEM((N_DEV - 1, m_per, n_per), jnp.int8),
            pltpu.VMEM((N_DEV - 1, m_per, n_per), jnp.int8),
            pltpu.VMEM((N_DEV - 1, 1, n_per), jnp.float32),
            pltpu.VMEM((N_DEV - 1, 1, n_per), jnp.float32),
            pltpu.SemaphoreType.DMA((2,)),
            pltpu.SemaphoreType.DMA((N_DEV - 1,)),
            pltpu.SemaphoreType.DMA((N_DEV - 1,)),
            pltpu.SemaphoreType.DMA((N_DEV - 1,)),
            pltpu.SemaphoreType.DMA((N_DEV - 1,)),
        ],
        compiler_params=pltpu.CompilerParams(
            collective_id=0,
            vmem_limit_bytes=100 * 1024 * 1024,
        ),
    )(x, w_mat, scale_x, scale_w)


# device time: 37551 ns/iter; 1.4046x vs baseline; 1.0024x over previous
import jax
import jax.numpy as jnp
from jax import lax
from jax.experimental import pallas as pl
from jax.experimental.pallas import tpu as pltpu

N_DEV = 8


def kernel(x, w_mat, scale_x, scale_w):
    m_per, k = x.shape
    n = w_mat.shape[1]
    n_per = n // N_DEV
    m_tot = m_per * N_DEV

    def body(x_ref, w_ref, sx_ref, sw_ref, out_ref,
             xb_ref, wv_ref, send_ref, comm_ref, ssc_ref, rsc_ref,
             wdma_sems, send_sems, recv_sems, ssc_sems, rsc_sems):
        my = lax.axis_index("i")

        barrier = pltpu.get_barrier_semaphore()
        for p in range(N_DEV):
            @pl.when(my != p)
            def _():
                pl.semaphore_signal(
                    barrier, inc=1, device_id=(p,),
                    device_id_type=pl.DeviceIdType.MESH,
                )
        pl.semaphore_wait(barrier, N_DEV - 1)

        s = sx_ref[0] * sw_ref[0]

        col = [lax.rem(my + 1 + st, N_DEV) * n_per for st in range(N_DEV)]

        def wcopy(st, buf):
            return pltpu.make_async_copy(
                w_ref.at[:, pl.ds(col[st], n_per)],
                wv_ref.at[buf],
                wdma_sems.at[buf],
            )

        wcopy(0, 0).start()
        xb_ref[...] = x_ref[...].astype(jnp.bfloat16)

        rdmas = []

        def drain(st):
            data, scales = rdmas[st]
            data.wait_recv()
            scales.wait_recv()
            srcdev = lax.rem(my - 1 - st + N_DEV, N_DEV)
            deq = comm_ref[st].astype(jnp.float32) * rsc_ref[st]
            out_ref[pl.ds(srcdev * m_per, m_per), :] = deq

        for st in range(N_DEV):
            buf = st % 2
            if st + 1 < N_DEV:
                wcopy(st + 1, 1 - buf).start()
            wcopy(st, buf).wait()
            wb = wv_ref[buf].astype(jnp.bfloat16)
            acc = jnp.dot(xb_ref[...], wb, preferred_element_type=jnp.float32)
            y = acc * s
            y = y * (0.5 * jnp.tanh(0.5 * y) + 0.5)

            if st == N_DEV - 1:
                out_ref[pl.ds(my * m_per, m_per), :] = y
                break

            j = lax.rem(my + 1 + st, N_DEV)
            amax = jnp.maximum(jnp.max(jnp.abs(y), axis=0, keepdims=True),
                               1e-30)
            send_ref[st] = jnp.round(y * (127.0 / amax)).astype(jnp.int8)
            ssc_ref[st] = amax * (1.0 / 127.0)
            data = pltpu.make_async_remote_copy(
                src_ref=send_ref.at[st],
                dst_ref=comm_ref.at[st],
                send_sem=send_sems.at[st],
                recv_sem=recv_sems.at[st],
                device_id=(j,),
                device_id_type=pl.DeviceIdType.MESH,
            )
            scales = pltpu.make_async_remote_copy(
                src_ref=ssc_ref.at[st],
                dst_ref=rsc_ref.at[st],
                send_sem=ssc_sems.at[st],
                recv_sem=rsc_sems.at[st],
                device_id=(j,),
                device_id_type=pl.DeviceIdType.MESH,
            )
            data.start()
            scales.start()
            rdmas.append((data, scales))

            if st >= 2:
                drain(st - 2)

        for st in range(N_DEV - 3, N_DEV - 1):
            drain(st)

        for st in range(N_DEV - 1):
            data, scales = rdmas[st]
            data.wait_send()
            scales.wait_send()

    return pl.pallas_call(
        body,
        out_shape=jax.ShapeDtypeStruct((m_tot, n_per), jnp.float32),
        in_specs=[
            pl.BlockSpec(memory_space=pltpu.VMEM),
            pl.BlockSpec(memory_space=pltpu.MemorySpace.HBM),
            pl.BlockSpec(memory_space=pltpu.SMEM),
            pl.BlockSpec(memory_space=pltpu.SMEM),
        ],
        out_specs=pl.BlockSpec(memory_space=pltpu.VMEM),
        scratch_shapes=[
            pltpu.VMEM((m_per, k), jnp.bfloat16),
            pltpu.VMEM((2, k, n_per), jnp.float32),
            pltpu.VMEM((N_DEV - 1, m_per, n_per), jnp.int8),
            pltpu.VMEM((N_DEV - 1, m_per, n_per), jnp.int8),
            pltpu.VMEM((N_DEV - 1, 1, n_per), jnp.float32),
            pltpu.VMEM((N_DEV - 1, 1, n_per), jnp.float32),
            pltpu.SemaphoreType.DMA((2,)),
            pltpu.SemaphoreType.DMA((N_DEV - 1,)),
            pltpu.SemaphoreType.DMA((N_DEV - 1,)),
            pltpu.SemaphoreType.DMA((N_DEV - 1,)),
            pltpu.SemaphoreType.DMA((N_DEV - 1,)),
        ],
        compiler_params=pltpu.CompilerParams(
            collective_id=0,
            vmem_limit_bytes=100 * 1024 * 1024,
        ),
    )(x, w_mat, scale_x, scale_w)


# device time: 37546 ns/iter; 1.4048x vs baseline; 1.0001x over previous
import jax
import jax.numpy as jnp
from jax import lax
from jax.experimental import pallas as pl
from jax.experimental.pallas import tpu as pltpu

N_DEV = 8


def kernel(x, w_mat, scale_x, scale_w):
    m_per, k = x.shape
    n = w_mat.shape[1]
    n_per = n // N_DEV
    m_tot = m_per * N_DEV

    def body(x_ref, w_ref, sx_ref, sw_ref, out_ref,
             xb_ref, wv_ref, send_ref, comm_ref, ssc_ref, rsc_ref,
             wdma_sems, send_sems, recv_sems, ssc_sems, rsc_sems):
        my = lax.axis_index("i")

        barrier = pltpu.get_barrier_semaphore()
        for p in range(N_DEV):
            @pl.when(my != p)
            def _():
                pl.semaphore_signal(
                    barrier, inc=1, device_id=(p,),
                    device_id_type=pl.DeviceIdType.MESH,
                )
        pl.semaphore_wait(barrier, N_DEV - 1)

        s = sx_ref[0] * sw_ref[0]

        col = [lax.rem(my + 1 + st, N_DEV) * n_per for st in range(N_DEV)]

        def wcopy(st, buf):
            return pltpu.make_async_copy(
                w_ref.at[:, pl.ds(col[st], n_per)],
                wv_ref.at[buf],
                wdma_sems.at[buf],
            )

        for st in range(N_DEV):
            wcopy(st, st).start()
        xb_ref[...] = x_ref[...].astype(jnp.bfloat16)

        rdmas = []

        def drain(st):
            data, scales = rdmas[st]
            data.wait_recv()
            scales.wait_recv()
            srcdev = lax.rem(my - 1 - st + N_DEV, N_DEV)
            deq = comm_ref[st].astype(jnp.float32) * rsc_ref[st]
            out_ref[pl.ds(srcdev * m_per, m_per), :] = deq

        for st in range(N_DEV):
            wcopy(st, st).wait()
            wb = wv_ref[st].astype(jnp.bfloat16)
            acc = jnp.dot(xb_ref[...], wb, preferred_element_type=jnp.float32)
            y = acc * s
            y = y * (0.5 * jnp.tanh(0.5 * y) + 0.5)

            if st == N_DEV - 1:
                out_ref[pl.ds(my * m_per, m_per), :] = y
                break

            j = lax.rem(my + 1 + st, N_DEV)
            amax = jnp.maximum(jnp.max(jnp.abs(y), axis=0, keepdims=True),
                               1e-30)
            send_ref[st] = jnp.round(y * (127.0 / amax)).astype(jnp.int8)
            ssc_ref[st] = amax * (1.0 / 127.0)
            data = pltpu.make_async_remote_copy(
                src_ref=send_ref.at[st],
                dst_ref=comm_ref.at[st],
                send_sem=send_sems.at[st],
                recv_sem=recv_sems.at[st],
                device_id=(j,),
                device_id_type=pl.DeviceIdType.MESH,
            )
            scales = pltpu.make_async_remote_copy(
                src_ref=ssc_ref.at[st],
                dst_ref=rsc_ref.at[st],
                send_sem=ssc_sems.at[st],
                recv_sem=rsc_sems.at[st],
                device_id=(j,),
                device_id_type=pl.DeviceIdType.MESH,
            )
            data.start()
            scales.start()
            rdmas.append((data, scales))

            if st >= 2:
                drain(st - 2)

        for st in range(N_DEV - 3, N_DEV - 1):
            drain(st)

        for st in range(N_DEV - 1):
            data, scales = rdmas[st]
            data.wait_send()
            scales.wait_send()

    return pl.pallas_call(
        body,
        out_shape=jax.ShapeDtypeStruct((m_tot, n_per), jnp.float32),
        in_specs=[
            pl.BlockSpec(memory_space=pltpu.VMEM),
            pl.BlockSpec(memory_space=pltpu.MemorySpace.HBM),
            pl.BlockSpec(memory_space=pltpu.SMEM),
            pl.BlockSpec(memory_space=pltpu.SMEM),
        ],
        out_specs=pl.BlockSpec(memory_space=pltpu.VMEM),
        scratch_shapes=[
            pltpu.VMEM((m_per, k), jnp.bfloat16),
            pltpu.VMEM((N_DEV, k, n_per), jnp.float32),
            pltpu.VMEM((N_DEV - 1, m_per, n_per), jnp.int8),
            pltpu.VMEM((N_DEV - 1, m_per, n_per), jnp.int8),
            pltpu.VMEM((N_DEV - 1, 1, n_per), jnp.float32),
            pltpu.VMEM((N_DEV - 1, 1, n_per), jnp.float32),
            pltpu.SemaphoreType.DMA((N_DEV,)),
            pltpu.SemaphoreType.DMA((N_DEV - 1,)),
            pltpu.SemaphoreType.DMA((N_DEV - 1,)),
            pltpu.SemaphoreType.DMA((N_DEV - 1,)),
            pltpu.SemaphoreType.DMA((N_DEV - 1,)),
        ],
        compiler_params=pltpu.CompilerParams(
            collective_id=0,
            vmem_limit_bytes=100 * 1024 * 1024,
        ),
    )(x, w_mat, scale_x, scale_w)
